# baseline (speedup 1.0000x reference)
import functools

import jax
import jax.numpy as jnp
from jax import lax
from jax.experimental import pallas as pl
from jax.experimental.pallas import tpu as pltpu
from jax.experimental.pallas import tpu_sc as plsc

N_NODES = 10000
N_EDGES = 320000
D = 128

NC = 2
NS = 16
NW = NC * NS
K = 128
NCHUNK = 80
EPT = NCHUNK * K
N_EPAD = NW * EPT
N_PAD = 10112
RPT = N_PAD // NS
RSTG = 128
STG_CHUNKS = (128, 128, 128, 128, 120)


def _sc_body(x_hbm, src_hbm, dst_hbm, out_hbm,
             src_v0, src_v1, dst_v0, dst_v1, rows_0, rows_1, stg_v, acc,
             isem_0, isem_1, gsem_0, gsem_1):
    cid = lax.axis_index("c")
    sid = lax.axis_index("s")
    wid = sid * NC + cid
    ebase = wid * EPT

    src_v = (src_v0, src_v1)
    dst_v = (dst_v0, dst_v1)
    rows = (rows_0, rows_1)
    isem = (isem_0, isem_1)
    gsem = (gsem_0, gsem_1)

    zeros16 = jnp.zeros((16,), jnp.float32)

    def zrow(i, c):
        for j in range(D // 16):
            stg_v[i, pl.ds(j * 16, 16)] = zeros16
        return c

    lax.fori_loop(0, RSTG, zrow, 0)
    off = 0
    for n in STG_CHUNKS:
        pltpu.sync_copy(stg_v.at[pl.ds(0, n)],
                        acc.at[pl.ds(sid * RPT + off, n)])
        off += n
    plsc.subcore_barrier()

    def istart(j, b):
        pltpu.async_copy(src_hbm.at[pl.ds(ebase + j * K, K)], src_v[b], isem[b])
        pltpu.async_copy(dst_hbm.at[pl.ds(ebase + j * K, K)], dst_v[b], isem[b])

    def iwait(b):
        pltpu.make_async_copy(src_hbm.at[pl.ds(0, K)], src_v[b], isem[b]).wait()
        pltpu.make_async_copy(dst_hbm.at[pl.ds(0, K)], dst_v[b], isem[b]).wait()

    def gather(b):
        pltpu.async_copy(x_hbm.at[src_v[b]], rows[b], gsem[b])

    def gwait(b):
        pltpu.make_async_copy(x_hbm.at[src_v[b]], rows[b], gsem[b]).wait()

    def scat(b):
        pltpu.sync_copy(rows[b], acc.at[dst_v[b]], add=True)

    istart(0, 0)
    iwait(0)
    gather(0)
    istart(1, 1)

    def body(p, c):
        j = 2 * p + 1
        gwait(0)
        iwait(1)
        gather(1)
        scat(0)
        istart(j + 1, 0)
        gwait(1)
        iwait(0)
        gather(0)
        scat(1)
        istart(j + 2, 1)
        return c

    lax.fori_loop(0, NCHUNK // 2 - 1, body, 0)
    gwait(0)
    iwait(1)
    gather(1)
    scat(0)
    gwait(1)
    scat(1)
    plsc.subcore_barrier()

    off = 0
    for n in STG_CHUNKS:
        r0 = sid * RPT + off
        pltpu.sync_copy(acc.at[pl.ds(r0, n)], stg_v.at[pl.ds(0, n)])
        pltpu.sync_copy(stg_v.at[pl.ds(0, n)], out_hbm.at[cid, pl.ds(r0, n)])
        off += n


_sc_scatter = functools.partial(
    pl.kernel,
    out_type=jax.ShapeDtypeStruct((NC, N_PAD, D), jnp.float32),
    mesh=plsc.VectorSubcoreMesh(core_axis_name="c", subcore_axis_name="s"),
    scratch_types=[
        pltpu.VMEM((K,), jnp.int32),
        pltpu.VMEM((K,), jnp.int32),
        pltpu.VMEM((K,), jnp.int32),
        pltpu.VMEM((K,), jnp.int32),
        pltpu.VMEM((K, D), jnp.float32),
        pltpu.VMEM((K, D), jnp.float32),
        pltpu.VMEM((RSTG, D), jnp.float32),
        pltpu.VMEM_SHARED((N_PAD, D), jnp.float32),
        pltpu.SemaphoreType.DMA,
        pltpu.SemaphoreType.DMA,
        pltpu.SemaphoreType.DMA,
        pltpu.SemaphoreType.DMA,
    ],
)(_sc_body)


ROWS_BLK = 1000


def _tc_body(p_ref, w_ref, o_ref):
    s = p_ref[0] + p_ref[1]
    o_ref[...] = jnp.maximum(
        lax.dot_general(s, w_ref[...], (((1,), (1,)), ((), ())),
                        preferred_element_type=jnp.float32),
        0.0)


def _combine(partials, W):
    return pl.pallas_call(
        _tc_body,
        grid=(N_NODES // ROWS_BLK,),
        in_specs=[
            pl.BlockSpec((NC, ROWS_BLK, D), lambda i: (0, i, 0)),
            pl.BlockSpec((D, D), lambda i: (0, 0)),
        ],
        out_specs=pl.BlockSpec((ROWS_BLK, D), lambda i: (i, 0)),
        out_shape=jax.ShapeDtypeStruct((N_NODES, D), jnp.float32),
    )(partials, W)


def kernel(x, edge_index, W):
    npad = N_EPAD - N_EDGES
    src = jnp.concatenate([edge_index[0], jnp.zeros((npad,), jnp.int32)])
    dst = jnp.concatenate(
        [edge_index[1], jnp.full((npad,), N_PAD - 1, jnp.int32)])
    partials = _sc_scatter(x, src, dst)
    return _combine(partials, W)


# speedup vs baseline: 3.3339x; 3.3339x over previous
import functools

import jax
import jax.numpy as jnp
from jax import lax
from jax.experimental import pallas as pl
from jax.experimental.pallas import tpu as pltpu
from jax.experimental.pallas import tpu_sc as plsc

N_NODES = 10000
N_EDGES = 320000
D = 128

NC = 2
NS = 16
NW = NC * NS
EPT = N_EDGES // NW
K = 128
NCHUNK = EPT // K
KTAIL = EPT - NCHUNK * K
N_PAD = 10112
RPT = N_PAD // NS
RSTG = 128
STG_CHUNKS = (128, 128, 128, 128, 120)


def _sc_body(x_hbm, src_hbm, dst_hbm, out_hbm,
             src_v0, src_v1, dst_v0, dst_v1, src_t, dst_t, rows_0, rows_1,
             stg_v, acc, isem_0, isem_1, gsem_0, gsem_1):
    cid = lax.axis_index("c")
    sid = lax.axis_index("s")
    wid = sid * NC + cid
    ebase = wid * EPT

    src_v = (src_v0, src_v1)
    dst_v = (dst_v0, dst_v1)
    rows = (rows_0, rows_1)
    isem = (isem_0, isem_1)
    gsem = (gsem_0, gsem_1)

    zeros16 = jnp.zeros((16,), jnp.float32)

    def zrow(i, c):
        for j in range(D // 16):
            stg_v[i, pl.ds(j * 16, 16)] = zeros16
        return c

    lax.fori_loop(0, RSTG, zrow, 0)
    off = 0
    for n in STG_CHUNKS:
        pltpu.sync_copy(stg_v.at[pl.ds(0, n)],
                        acc.at[pl.ds(sid * RPT + off, n)])
        off += n
    plsc.subcore_barrier()

    def istart(j, b):
        pltpu.async_copy(src_hbm.at[pl.ds(ebase + j * K, K)], src_v[b], isem[b])
        pltpu.async_copy(dst_hbm.at[pl.ds(ebase + j * K, K)], dst_v[b], isem[b])

    def iwait(b):
        pltpu.make_async_copy(src_hbm.at[pl.ds(0, K)], src_v[b], isem[b]).wait()
        pltpu.make_async_copy(dst_hbm.at[pl.ds(0, K)], dst_v[b], isem[b]).wait()

    def gather(b):
        pltpu.async_copy(x_hbm.at[src_v[b]], rows[b], gsem[b])

    def gwait(b):
        pltpu.make_async_copy(x_hbm.at[src_v[b]], rows[b], gsem[b]).wait()

    def scat(b):
        pltpu.sync_copy(rows[b], acc.at[dst_v[b]], add=True)

    istart(0, 0)
    iwait(0)
    gather(0)
    istart(1, 1)

    def body(p, c):
        j = 2 * p + 1
        gwait(0)
        iwait(1)
        gather(1)
        scat(0)
        istart(j + 1, 0)
        gwait(1)
        iwait(0)
        gather(0)
        scat(1)
        istart(j + 2, 1)
        return c

    lax.fori_loop(0, NCHUNK // 2 - 1, body, 0)
    gwait(0)
    iwait(1)
    gather(1)
    scat(0)
    gwait(1)
    scat(1)
    tbase = ebase + NCHUNK * K
    pltpu.sync_copy(src_hbm.at[pl.ds(tbase, KTAIL)], src_t)
    pltpu.sync_copy(dst_hbm.at[pl.ds(tbase, KTAIL)], dst_t)
    tr = rows_0.at[pl.ds(0, KTAIL)]
    pltpu.async_copy(x_hbm.at[src_t], tr, gsem_0).wait()
    pltpu.sync_copy(tr, acc.at[dst_t], add=True)
    plsc.subcore_barrier()

    off = 0
    for n in STG_CHUNKS:
        r0 = sid * RPT + off
        pltpu.sync_copy(acc.at[pl.ds(r0, n)], stg_v.at[pl.ds(0, n)])
        pltpu.sync_copy(stg_v.at[pl.ds(0, n)], out_hbm.at[cid, pl.ds(r0, n)])
        off += n


_sc_scatter = functools.partial(
    pl.kernel,
    out_type=jax.ShapeDtypeStruct((NC, N_PAD, D), jnp.float32),
    mesh=plsc.VectorSubcoreMesh(core_axis_name="c", subcore_axis_name="s"),
    scratch_types=[
        pltpu.VMEM((K,), jnp.int32),
        pltpu.VMEM((K,), jnp.int32),
        pltpu.VMEM((K,), jnp.int32),
        pltpu.VMEM((K,), jnp.int32),
        pltpu.VMEM((KTAIL,), jnp.int32),
        pltpu.VMEM((KTAIL,), jnp.int32),
        pltpu.VMEM((K, D), jnp.float32),
        pltpu.VMEM((K, D), jnp.float32),
        pltpu.VMEM((RSTG, D), jnp.float32),
        pltpu.VMEM_SHARED((N_PAD, D), jnp.float32),
        pltpu.SemaphoreType.DMA,
        pltpu.SemaphoreType.DMA,
        pltpu.SemaphoreType.DMA,
        pltpu.SemaphoreType.DMA,
    ],
)(_sc_body)


ROWS_BLK = 1000


def _tc_body(p_ref, w_ref, o_ref):
    s = p_ref[0] + p_ref[1]
    o_ref[...] = jnp.maximum(
        lax.dot_general(s, w_ref[...], (((1,), (1,)), ((), ())),
                        preferred_element_type=jnp.float32),
        0.0)


def _combine(partials, W):
    return pl.pallas_call(
        _tc_body,
        grid=(N_NODES // ROWS_BLK,),
        in_specs=[
            pl.BlockSpec((NC, ROWS_BLK, D), lambda i: (0, i, 0)),
            pl.BlockSpec((D, D), lambda i: (0, 0)),
        ],
        out_specs=pl.BlockSpec((ROWS_BLK, D), lambda i: (i, 0)),
        out_shape=jax.ShapeDtypeStruct((N_NODES, D), jnp.float32),
    )(partials, W)


def kernel(x, edge_index, W):
    partials = _sc_scatter(x, edge_index[0], edge_index[1])
    return _combine(partials, W)


# speedup vs baseline: 3.3370x; 1.0009x over previous
import functools

import jax
import jax.numpy as jnp
from jax import lax
from jax.experimental import pallas as pl
from jax.experimental.pallas import tpu as pltpu
from jax.experimental.pallas import tpu_sc as plsc

N_NODES = 10000
N_EDGES = 320000
D = 128

NC = 2
NS = 16
NW = NC * NS
EPT = N_EDGES // NW
K = 128
NCHUNK = EPT // K
KTAIL = EPT - NCHUNK * K
N_PAD = 10112
RPT = N_PAD // NS
ZCH = (128, 128, 128, 128, 120)


def _sc_body(x_hbm, src_hbm, dst_hbm, out_hbm,
             src_v0, src_v1, src_v2, dst_v0, dst_v1, dst_v2, src_t, dst_t,
             rows_0, rows_1, rows_2, acc,
             isem_0, isem_1, isem_2, gsem_0, gsem_1, gsem_2,
             ssem_0, ssem_1, ssem_2, wsem):
    cid = lax.axis_index("c")
    sid = lax.axis_index("s")
    wid = sid * NC + cid
    ebase = wid * EPT

    src_v = (src_v0, src_v1, src_v2)
    dst_v = (dst_v0, dst_v1, dst_v2)
    rows = (rows_0, rows_1, rows_2)
    isem = (isem_0, isem_1, isem_2)
    gsem = (gsem_0, gsem_1, gsem_2)
    ssem = (ssem_0, ssem_1, ssem_2)

    zeros16 = jnp.zeros((16,), jnp.float32)

    def zrow(i, c):
        for j in range(D // 16):
            rows_0[i, pl.ds(j * 16, 16)] = zeros16
        return c

    lax.fori_loop(0, K, zrow, 0)
    off = 0
    for n in ZCH:
        pltpu.async_copy(rows_0.at[pl.ds(0, n)],
                         acc.at[pl.ds(sid * RPT + off, n)], wsem)
        off += n
    off = 0
    for n in ZCH:
        pltpu.make_async_copy(rows_0.at[pl.ds(0, n)],
                              acc.at[pl.ds(sid * RPT + off, n)], wsem).wait()
        off += n
    plsc.subcore_barrier()

    def istart(j, b):
        pltpu.async_copy(src_hbm.at[pl.ds(ebase + j * K, K)], src_v[b], isem[b])
        pltpu.async_copy(dst_hbm.at[pl.ds(ebase + j * K, K)], dst_v[b], isem[b])

    def iwait(b):
        pltpu.make_async_copy(src_hbm.at[pl.ds(0, K)], src_v[b], isem[b]).wait()
        pltpu.make_async_copy(dst_hbm.at[pl.ds(0, K)], dst_v[b], isem[b]).wait()

    def gather(b):
        pltpu.async_copy(x_hbm.at[src_v[b]], rows[b], gsem[b])

    def gwait(b):
        pltpu.make_async_copy(x_hbm.at[src_v[b]], rows[b], gsem[b]).wait()

    def sstart(b):
        pltpu.async_copy(rows[b], acc.at[dst_v[b]], ssem[b], add=True)

    def swait(b):
        pltpu.make_async_copy(rows[b], acc.at[dst_v[b]], ssem[b]).wait()

    istart(0, 0)
    istart(1, 1)
    iwait(0)
    gather(0)
    gwait(0)
    sstart(0)
    istart(2, 2)
    iwait(1)
    gather(1)

    def body(q, c):
        for r in range(3):
            t = 3 * q + 1 + r
            a = (1 + r) % 3
            g = (2 + r) % 3
            i = r
            gwait(a)
            sstart(a)
            swait(i)
            istart(t + 2, i)
            iwait(g)
            gather(g)
        return c

    lax.fori_loop(0, (NCHUNK - 3) // 3, body, 0)
    gwait(1)
    sstart(1)
    swait(0)
    iwait(2)
    gather(2)
    gwait(2)
    sstart(2)
    swait(1)
    swait(2)
    tbase = ebase + NCHUNK * K
    pltpu.sync_copy(src_hbm.at[pl.ds(tbase, KTAIL)], src_t)
    pltpu.sync_copy(dst_hbm.at[pl.ds(tbase, KTAIL)], dst_t)
    tr = rows_0.at[pl.ds(0, KTAIL)]
    pltpu.async_copy(x_hbm.at[src_t], tr, gsem_0).wait()
    pltpu.sync_copy(tr, acc.at[dst_t], add=True)
    plsc.subcore_barrier()

    r0 = sid * RPT
    pltpu.async_copy(acc.at[pl.ds(r0, RPT)],
                     out_hbm.at[cid, pl.ds(r0, RPT)], wsem)
    pltpu.make_async_copy(acc.at[pl.ds(r0, RPT)],
                          out_hbm.at[cid, pl.ds(r0, RPT)], wsem).wait()


_sc_scatter = functools.partial(
    pl.kernel,
    out_type=jax.ShapeDtypeStruct((NC, N_PAD, D), jnp.float32),
    mesh=plsc.VectorSubcoreMesh(core_axis_name="c", subcore_axis_name="s"),
    scratch_types=[
        pltpu.VMEM((K,), jnp.int32),
        pltpu.VMEM((K,), jnp.int32),
        pltpu.VMEM((K,), jnp.int32),
        pltpu.VMEM((K,), jnp.int32),
        pltpu.VMEM((K,), jnp.int32),
        pltpu.VMEM((K,), jnp.int32),
        pltpu.VMEM((KTAIL,), jnp.int32),
        pltpu.VMEM((KTAIL,), jnp.int32),
        pltpu.VMEM((K, D), jnp.float32),
        pltpu.VMEM((K, D), jnp.float32),
        pltpu.VMEM((K, D), jnp.float32),
        pltpu.VMEM_SHARED((N_PAD, D), jnp.float32),
        pltpu.SemaphoreType.DMA,
        pltpu.SemaphoreType.DMA,
        pltpu.SemaphoreType.DMA,
        pltpu.SemaphoreType.DMA,
        pltpu.SemaphoreType.DMA,
        pltpu.SemaphoreType.DMA,
        pltpu.SemaphoreType.DMA,
        pltpu.SemaphoreType.DMA,
        pltpu.SemaphoreType.DMA,
        pltpu.SemaphoreType.DMA,
    ],
)(_sc_body)


ROWS_BLK = 1000


def _tc_body(p_ref, w_ref, o_ref):
    s = p_ref[0] + p_ref[1]
    o_ref[...] = jnp.maximum(
        lax.dot_general(s, w_ref[...], (((1,), (1,)), ((), ())),
                        preferred_element_type=jnp.float32),
        0.0)


def _combine(partials, W):
    return pl.pallas_call(
        _tc_body,
        grid=(N_NODES // ROWS_BLK,),
        in_specs=[
            pl.BlockSpec((NC, ROWS_BLK, D), lambda i: (0, i, 0)),
            pl.BlockSpec((D, D), lambda i: (0, 0)),
        ],
        out_specs=pl.BlockSpec((ROWS_BLK, D), lambda i: (i, 0)),
        out_shape=jax.ShapeDtypeStruct((N_NODES, D), jnp.float32),
    )(partials, W)


def kernel(x, edge_index, W):
    partials = _sc_scatter(x, edge_index[0], edge_index[1])
    return _combine(partials, W)


# speedup vs baseline: 3.6185x; 1.0844x over previous
import functools

import jax
import jax.numpy as jnp
from jax import lax
from jax.experimental import pallas as pl
from jax.experimental.pallas import tpu as pltpu
from jax.experimental.pallas import tpu_sc as plsc

N_NODES = 10000
N_EDGES = 320000
D = 128

NC = 2
NS = 16
NW = NC * NS
EPT = N_EDGES // NW
K = 128
NCHUNK = EPT // K
KTAIL = EPT - NCHUNK * K
N_PAD = 10112
RPT = N_PAD // NS
ZCH = (128, 128, 128, 128, 120)


def _sc_body(x_hbm, ei_hbm, out_hbm,
             src_v0, src_v1, src_v2, dst_v0, dst_v1, dst_v2, src_t, dst_t,
             rows_0, rows_1, rows_2, acc,
             isem_0, isem_1, isem_2, gsem_0, gsem_1, gsem_2,
             ssem_0, ssem_1, ssem_2, wsem):
    cid = lax.axis_index("c")
    sid = lax.axis_index("s")
    wid = sid * NC + cid
    ebase = wid * EPT
    dbase = N_EDGES + ebase

    src_v = (src_v0, src_v1, src_v2)
    dst_v = (dst_v0, dst_v1, dst_v2)
    rows = (rows_0, rows_1, rows_2)
    isem = (isem_0, isem_1, isem_2)
    gsem = (gsem_0, gsem_1, gsem_2)
    ssem = (ssem_0, ssem_1, ssem_2)

    zeros16 = jnp.zeros((16,), jnp.float32)

    def zrow(i, c):
        for j in range(D // 16):
            rows_0[i, pl.ds(j * 16, 16)] = zeros16
        return c

    lax.fori_loop(0, K, zrow, 0)
    off = 0
    for n in ZCH:
        pltpu.async_copy(rows_0.at[pl.ds(0, n)],
                         acc.at[pl.ds(sid * RPT + off, n)], wsem)
        off += n
    off = 0
    for n in ZCH:
        pltpu.make_async_copy(rows_0.at[pl.ds(0, n)],
                              acc.at[pl.ds(sid * RPT + off, n)], wsem).wait()
        off += n
    plsc.subcore_barrier()

    def istart(j, b):
        pltpu.async_copy(ei_hbm.at[pl.ds(ebase + j * K, K)], src_v[b], isem[b])
        pltpu.async_copy(ei_hbm.at[pl.ds(dbase + j * K, K)], dst_v[b], isem[b])

    def iwait(b):
        pltpu.make_async_copy(ei_hbm.at[pl.ds(0, K)], src_v[b], isem[b]).wait()
        pltpu.make_async_copy(ei_hbm.at[pl.ds(0, K)], dst_v[b], isem[b]).wait()

    def gather(b):
        pltpu.async_copy(x_hbm.at[src_v[b]], rows[b], gsem[b])

    def gwait(b):
        pltpu.make_async_copy(x_hbm.at[src_v[b]], rows[b], gsem[b]).wait()

    def sstart(b):
        pltpu.async_copy(rows[b], acc.at[dst_v[b]], ssem[b], add=True)

    def swait(b):
        pltpu.make_async_copy(rows[b], acc.at[dst_v[b]], ssem[b]).wait()

    istart(0, 0)
    istart(1, 1)
    iwait(0)
    gather(0)
    gwait(0)
    sstart(0)
    istart(2, 2)
    iwait(1)
    gather(1)

    def body(q, c):
        for r in range(3):
            t = 3 * q + 1 + r
            a = (1 + r) % 3
            g = (2 + r) % 3
            i = r
            gwait(a)
            sstart(a)
            swait(i)
            istart(t + 2, i)
            iwait(g)
            gather(g)
        return c

    lax.fori_loop(0, (NCHUNK - 3) // 3, body, 0)
    gwait(1)
    sstart(1)
    swait(0)
    iwait(2)
    gather(2)
    gwait(2)
    sstart(2)
    swait(1)
    swait(2)
    tbase = NCHUNK * K
    pltpu.sync_copy(ei_hbm.at[pl.ds(ebase + tbase, KTAIL)], src_t)
    pltpu.sync_copy(ei_hbm.at[pl.ds(dbase + tbase, KTAIL)], dst_t)
    tr = rows_0.at[pl.ds(0, KTAIL)]
    pltpu.async_copy(x_hbm.at[src_t], tr, gsem_0).wait()
    pltpu.sync_copy(tr, acc.at[dst_t], add=True)
    plsc.subcore_barrier()

    r0 = sid * RPT
    pltpu.async_copy(acc.at[pl.ds(r0, RPT)],
                     out_hbm.at[cid, pl.ds(r0, RPT)], wsem)
    pltpu.make_async_copy(acc.at[pl.ds(r0, RPT)],
                          out_hbm.at[cid, pl.ds(r0, RPT)], wsem).wait()


_sc_scatter = functools.partial(
    pl.kernel,
    out_type=jax.ShapeDtypeStruct((NC, N_PAD, D), jnp.float32),
    mesh=plsc.VectorSubcoreMesh(core_axis_name="c", subcore_axis_name="s"),
    scratch_types=[
        pltpu.VMEM((K,), jnp.int32),
        pltpu.VMEM((K,), jnp.int32),
        pltpu.VMEM((K,), jnp.int32),
        pltpu.VMEM((K,), jnp.int32),
        pltpu.VMEM((K,), jnp.int32),
        pltpu.VMEM((K,), jnp.int32),
        pltpu.VMEM((KTAIL,), jnp.int32),
        pltpu.VMEM((KTAIL,), jnp.int32),
        pltpu.VMEM((K, D), jnp.float32),
        pltpu.VMEM((K, D), jnp.float32),
        pltpu.VMEM((K, D), jnp.float32),
        pltpu.VMEM_SHARED((N_PAD, D), jnp.float32),
        pltpu.SemaphoreType.DMA,
        pltpu.SemaphoreType.DMA,
        pltpu.SemaphoreType.DMA,
        pltpu.SemaphoreType.DMA,
        pltpu.SemaphoreType.DMA,
        pltpu.SemaphoreType.DMA,
        pltpu.SemaphoreType.DMA,
        pltpu.SemaphoreType.DMA,
        pltpu.SemaphoreType.DMA,
        pltpu.SemaphoreType.DMA,
    ],
)(_sc_body)


ROWS_BLK = 2000


def _tc_body(p_ref, w_ref, o_ref):
    s = p_ref[0] + p_ref[1]
    o_ref[...] = jnp.maximum(
        lax.dot_general(s, w_ref[...], (((1,), (1,)), ((), ())),
                        preferred_element_type=jnp.float32),
        0.0)


def _combine(partials, W):
    return pl.pallas_call(
        _tc_body,
        grid=(N_NODES // ROWS_BLK,),
        in_specs=[
            pl.BlockSpec((NC, ROWS_BLK, D), lambda i: (0, i, 0)),
            pl.BlockSpec((D, D), lambda i: (0, 0)),
        ],
        out_specs=pl.BlockSpec((ROWS_BLK, D), lambda i: (i, 0)),
        out_shape=jax.ShapeDtypeStruct((N_NODES, D), jnp.float32),
    )(partials, W)


def kernel(x, edge_index, W):
    partials = _sc_scatter(x, edge_index.reshape(2 * N_EDGES))
    return _combine(partials, W)


# speedup vs baseline: 4.2524x; 1.1752x over previous
import functools

import jax
import jax.numpy as jnp
from jax import lax
from jax.experimental import pallas as pl
from jax.experimental.pallas import tpu as pltpu
from jax.experimental.pallas import tpu_sc as plsc

N_NODES = 10000
N_EDGES = 320000
D = 128

NC = 2
NS = 16
NW = NC * NS
EPT = N_EDGES // NW
K = 128
NCHUNK = EPT // K
KTAIL = EPT - NCHUNK * K
N_PAD = 10112
RPT = N_PAD // NS
ZCH = (128, 128, 128, 128, 120)


def _sc_body(x_hbm, ei_hbm, out_hbm,
             src_v0, src_v1, src_v2, dst_v0, dst_v1, dst_v2, src_t, dst_t,
             rows_0, rows_1, rows_2, acc,
             isem_0, isem_1, isem_2, gsem_0, gsem_1, gsem_2,
             ssem_0, ssem_1, ssem_2, wsem):
    cid = lax.axis_index("c")
    sid = lax.axis_index("s")
    wid = sid * NC + cid
    ebase = wid * EPT
    dbase = N_EDGES + ebase

    src_v = (src_v0, src_v1, src_v2)
    dst_v = (dst_v0, dst_v1, dst_v2)
    rows = (rows_0, rows_1, rows_2)
    isem = (isem_0, isem_1, isem_2)
    gsem = (gsem_0, gsem_1, gsem_2)
    ssem = (ssem_0, ssem_1, ssem_2)

    zeros16 = jnp.zeros((16,), jnp.float32)

    def zrow(i, c):
        for j in range(D // 16):
            rows_0[i, pl.ds(j * 16, 16)] = zeros16
        return c

    lax.fori_loop(0, K, zrow, 0)
    off = 0
    for n in ZCH:
        pltpu.async_copy(rows_0.at[pl.ds(0, n)],
                         acc.at[pl.ds(sid * RPT + off, n)], wsem)
        off += n
    off = 0
    for n in ZCH:
        pltpu.make_async_copy(rows_0.at[pl.ds(0, n)],
                              acc.at[pl.ds(sid * RPT + off, n)], wsem).wait()
        off += n
    plsc.subcore_barrier()

    def istart(j, b):
        pltpu.async_copy(ei_hbm.at[pl.ds(ebase + j * K, K)], src_v[b], isem[b])
        pltpu.async_copy(ei_hbm.at[pl.ds(dbase + j * K, K)], dst_v[b], isem[b])

    def iwait(b):
        pltpu.make_async_copy(ei_hbm.at[pl.ds(0, K)], src_v[b], isem[b]).wait()
        pltpu.make_async_copy(ei_hbm.at[pl.ds(0, K)], dst_v[b], isem[b]).wait()

    H = K // 2

    def gather(b):
        pltpu.async_copy(x_hbm.at[src_v[b].at[pl.ds(0, H)]],
                         rows[b].at[pl.ds(0, H)], gsem[b])
        pltpu.async_copy(x_hbm.at[src_v[b].at[pl.ds(H, H)]],
                         rows[b].at[pl.ds(H, H)], gsem[b])

    def gwait(b):
        pltpu.make_async_copy(x_hbm.at[src_v[b].at[pl.ds(0, H)]],
                              rows[b].at[pl.ds(0, H)], gsem[b]).wait()
        pltpu.make_async_copy(x_hbm.at[src_v[b].at[pl.ds(H, H)]],
                              rows[b].at[pl.ds(H, H)], gsem[b]).wait()

    def sstart(b):
        pltpu.async_copy(rows[b], acc.at[dst_v[b]], ssem[b], add=True)

    def swait(b):
        pltpu.make_async_copy(rows[b], acc.at[dst_v[b]], ssem[b]).wait()

    def scat(b):
        pltpu.sync_copy(rows[b], acc.at[dst_v[b]], add=True)

    istart(0, 0)
    istart(1, 1)
    istart(2, 2)
    iwait(0)
    gather(0)
    iwait(1)
    gather(1)

    def body(q, c):
        for r in range(3):
            t = 3 * q + r
            a = r
            n2 = (r + 2) % 3
            gwait(a)
            scat(a)
            istart(t + 3, a)
            iwait(n2)
            gather(n2)
        return c

    lax.fori_loop(0, (NCHUNK - 3) // 3, body, 0)
    gwait(0)
    scat(0)
    iwait(2)
    gather(2)
    gwait(1)
    scat(1)
    gwait(2)
    scat(2)
    tbase = NCHUNK * K
    pltpu.sync_copy(ei_hbm.at[pl.ds(ebase + tbase, KTAIL)], src_t)
    pltpu.sync_copy(ei_hbm.at[pl.ds(dbase + tbase, KTAIL)], dst_t)
    tr = rows_0.at[pl.ds(0, KTAIL)]
    pltpu.async_copy(x_hbm.at[src_t], tr, gsem_0).wait()
    pltpu.sync_copy(tr, acc.at[dst_t], add=True)
    plsc.subcore_barrier()

    r0 = sid * RPT
    pltpu.async_copy(acc.at[pl.ds(r0, RPT)],
                     out_hbm.at[cid, pl.ds(r0, RPT)], wsem)
    pltpu.make_async_copy(acc.at[pl.ds(r0, RPT)],
                          out_hbm.at[cid, pl.ds(r0, RPT)], wsem).wait()


_sc_scatter = functools.partial(
    pl.kernel,
    out_type=jax.ShapeDtypeStruct((NC, N_PAD, D), jnp.float32),
    mesh=plsc.VectorSubcoreMesh(core_axis_name="c", subcore_axis_name="s"),
    scratch_types=[
        pltpu.VMEM((K,), jnp.int32),
        pltpu.VMEM((K,), jnp.int32),
        pltpu.VMEM((K,), jnp.int32),
        pltpu.VMEM((K,), jnp.int32),
        pltpu.VMEM((K,), jnp.int32),
        pltpu.VMEM((K,), jnp.int32),
        pltpu.VMEM((KTAIL,), jnp.int32),
        pltpu.VMEM((KTAIL,), jnp.int32),
        pltpu.VMEM((K, D), jnp.float32),
        pltpu.VMEM((K, D), jnp.float32),
        pltpu.VMEM((K, D), jnp.float32),
        pltpu.VMEM_SHARED((N_PAD, D), jnp.float32),
        pltpu.SemaphoreType.DMA,
        pltpu.SemaphoreType.DMA,
        pltpu.SemaphoreType.DMA,
        pltpu.SemaphoreType.DMA,
        pltpu.SemaphoreType.DMA,
        pltpu.SemaphoreType.DMA,
        pltpu.SemaphoreType.DMA,
        pltpu.SemaphoreType.DMA,
        pltpu.SemaphoreType.DMA,
        pltpu.SemaphoreType.DMA,
    ],
)(_sc_body)


ROWS_BLK = 2000


def _tc_body(p_ref, w_ref, o_ref):
    s = p_ref[0] + p_ref[1]
    o_ref[...] = jnp.maximum(
        lax.dot_general(s, w_ref[...], (((1,), (1,)), ((), ())),
                        preferred_element_type=jnp.float32),
        0.0)


def _combine(partials, W):
    return pl.pallas_call(
        _tc_body,
        grid=(N_NODES // ROWS_BLK,),
        in_specs=[
            pl.BlockSpec((NC, ROWS_BLK, D), lambda i: (0, i, 0)),
            pl.BlockSpec((D, D), lambda i: (0, 0)),
        ],
        out_specs=pl.BlockSpec((ROWS_BLK, D), lambda i: (i, 0)),
        out_shape=jax.ShapeDtypeStruct((N_NODES, D), jnp.float32),
    )(partials, W)


def kernel(x, edge_index, W):
    partials = _sc_scatter(x, edge_index.reshape(2 * N_EDGES))
    return _combine(partials, W)


# speedup vs baseline: 4.2593x; 1.0016x over previous
import functools

import jax
import jax.numpy as jnp
from jax import lax
from jax.experimental import pallas as pl
from jax.experimental.pallas import tpu as pltpu
from jax.experimental.pallas import tpu_sc as plsc

N_NODES = 10000
N_EDGES = 320000
D = 128

NC = 2
NS = 16
NW = NC * NS
EPT = N_EDGES // NW
K = 128
NCHUNK = EPT // K
KTAIL = EPT - NCHUNK * K
N_PAD = 10112
RPT = N_PAD // NS
ZCH = (128, 128, 128, 128, 120)


def _sc_body(x_hbm, ei_hbm, out_hbm,
             src_v0, src_v1, src_v2, dst_v0, dst_v1, dst_v2, src_t, dst_t,
             rows_0, rows_1, rows_2, acc,
             isem_0, isem_1, isem_2, gsem_0, gsem_1, gsem_2,
             ssem_0, ssem_1, ssem_2, wsem):
    cid = lax.axis_index("c")
    sid = lax.axis_index("s")
    wid = sid * NC + cid
    ebase = wid * EPT
    dbase = N_EDGES + ebase

    src_v = (src_v0, src_v1, src_v2)
    dst_v = (dst_v0, dst_v1, dst_v2)
    rows = (rows_0, rows_1, rows_2)
    isem = (isem_0, isem_1, isem_2)
    gsem = (gsem_0, gsem_1, gsem_2)
    ssem = (ssem_0, ssem_1, ssem_2)

    zeros16 = jnp.zeros((16,), jnp.float32)

    def zrow(i, c):
        for j in range(D // 16):
            rows_0[i, pl.ds(j * 16, 16)] = zeros16
        return c

    lax.fori_loop(0, K, zrow, 0)

    def istart(j, b):
        pltpu.async_copy(ei_hbm.at[pl.ds(ebase + j * K, K)], src_v[b], isem[b])
        pltpu.async_copy(ei_hbm.at[pl.ds(dbase + j * K, K)], dst_v[b], isem[b])

    def iwait(b):
        pltpu.make_async_copy(ei_hbm.at[pl.ds(0, K)], src_v[b], isem[b]).wait()
        pltpu.make_async_copy(ei_hbm.at[pl.ds(0, K)], dst_v[b], isem[b]).wait()

    H = K // 2

    def gather(b):
        pltpu.async_copy(x_hbm.at[src_v[b].at[pl.ds(0, H)]],
                         rows[b].at[pl.ds(0, H)], gsem[b])
        pltpu.async_copy(x_hbm.at[src_v[b].at[pl.ds(H, H)]],
                         rows[b].at[pl.ds(H, H)], gsem[b])

    def gwait(b):
        pltpu.make_async_copy(x_hbm.at[src_v[b].at[pl.ds(0, H)]],
                              rows[b].at[pl.ds(0, H)], gsem[b]).wait()
        pltpu.make_async_copy(x_hbm.at[src_v[b].at[pl.ds(H, H)]],
                              rows[b].at[pl.ds(H, H)], gsem[b]).wait()

    def sstart(b):
        pltpu.async_copy(rows[b], acc.at[dst_v[b]], ssem[b], add=True)

    def swait(b):
        pltpu.make_async_copy(rows[b], acc.at[dst_v[b]], ssem[b]).wait()

    def scat(b):
        pltpu.sync_copy(rows[b], acc.at[dst_v[b]], add=True)

    istart(0, 0)
    istart(1, 1)
    istart(2, 2)
    off = 0
    for n in ZCH:
        pltpu.async_copy(rows_0.at[pl.ds(0, n)],
                         acc.at[pl.ds(sid * RPT + off, n)], wsem)
        off += n
    off = 0
    for n in ZCH:
        pltpu.make_async_copy(rows_0.at[pl.ds(0, n)],
                              acc.at[pl.ds(sid * RPT + off, n)], wsem).wait()
        off += n
    iwait(0)
    gather(0)
    iwait(1)
    gather(1)
    plsc.subcore_barrier()

    def body(q, c):
        for r in range(3):
            t = 3 * q + r
            a = r
            n2 = (r + 2) % 3
            gwait(a)
            scat(a)
            istart(t + 3, a)
            iwait(n2)
            gather(n2)
        return c

    lax.fori_loop(0, (NCHUNK - 3) // 3, body, 0)
    gwait(0)
    scat(0)
    iwait(2)
    gather(2)
    gwait(1)
    scat(1)
    gwait(2)
    scat(2)
    tbase = NCHUNK * K
    pltpu.sync_copy(ei_hbm.at[pl.ds(ebase + tbase, KTAIL)], src_t)
    pltpu.sync_copy(ei_hbm.at[pl.ds(dbase + tbase, KTAIL)], dst_t)
    tr = rows_0.at[pl.ds(0, KTAIL)]
    pltpu.async_copy(x_hbm.at[src_t], tr, gsem_0).wait()
    pltpu.sync_copy(tr, acc.at[dst_t], add=True)
    plsc.subcore_barrier()

    r0 = sid * RPT
    pltpu.async_copy(acc.at[pl.ds(r0, RPT)],
                     out_hbm.at[cid, pl.ds(r0, RPT)], wsem)
    pltpu.make_async_copy(acc.at[pl.ds(r0, RPT)],
                          out_hbm.at[cid, pl.ds(r0, RPT)], wsem).wait()


_sc_scatter = functools.partial(
    pl.kernel,
    out_type=jax.ShapeDtypeStruct((NC, N_PAD, D), jnp.float32),
    mesh=plsc.VectorSubcoreMesh(core_axis_name="c", subcore_axis_name="s"),
    scratch_types=[
        pltpu.VMEM((K,), jnp.int32),
        pltpu.VMEM((K,), jnp.int32),
        pltpu.VMEM((K,), jnp.int32),
        pltpu.VMEM((K,), jnp.int32),
        pltpu.VMEM((K,), jnp.int32),
        pltpu.VMEM((K,), jnp.int32),
        pltpu.VMEM((KTAIL,), jnp.int32),
        pltpu.VMEM((KTAIL,), jnp.int32),
        pltpu.VMEM((K, D), jnp.float32),
        pltpu.VMEM((K, D), jnp.float32),
        pltpu.VMEM((K, D), jnp.float32),
        pltpu.VMEM_SHARED((N_PAD, D), jnp.float32),
        pltpu.SemaphoreType.DMA,
        pltpu.SemaphoreType.DMA,
        pltpu.SemaphoreType.DMA,
        pltpu.SemaphoreType.DMA,
        pltpu.SemaphoreType.DMA,
        pltpu.SemaphoreType.DMA,
        pltpu.SemaphoreType.DMA,
        pltpu.SemaphoreType.DMA,
        pltpu.SemaphoreType.DMA,
        pltpu.SemaphoreType.DMA,
    ],
)(_sc_body)


ROWS_BLK = 2000


def _tc_body(p_ref, w_ref, o_ref):
    s = p_ref[0] + p_ref[1]
    o_ref[...] = jnp.maximum(
        lax.dot_general(s, w_ref[...], (((1,), (1,)), ((), ())),
                        preferred_element_type=jnp.float32),
        0.0)


def _combine(partials, W):
    return pl.pallas_call(
        _tc_body,
        grid=(N_NODES // ROWS_BLK,),
        in_specs=[
            pl.BlockSpec((NC, ROWS_BLK, D), lambda i: (0, i, 0)),
            pl.BlockSpec((D, D), lambda i: (0, 0)),
        ],
        out_specs=pl.BlockSpec((ROWS_BLK, D), lambda i: (i, 0)),
        out_shape=jax.ShapeDtypeStruct((N_NODES, D), jnp.float32),
    )(partials, W)


def kernel(x, edge_index, W):
    partials = _sc_scatter(x, edge_index.reshape(2 * N_EDGES))
    return _combine(partials, W)
